# baseline (speedup 1.0000x reference)
import functools

import jax
import jax.numpy as jnp
from jax.experimental import pallas as pl
from jax.experimental.pallas import tpu as pltpu


def _ceil_to(x, m):
    return ((x + m - 1) // m) * m


def _gcn_block_kernel(sim_ref, x_ref, w_ref, b_ref, o_ref):
    h = jnp.dot(sim_ref[...], x_ref[...], preferred_element_type=jnp.float32)
    y = jnp.dot(h, w_ref[...], preferred_element_type=jnp.float32)
    o_ref[...] = jnp.maximum(y + b_ref[...], 0.0)


@functools.partial(jax.jit, static_argnames=("block_rows",))
def _gcn_layer(similar, inputs, weight, bias, *, block_rows=1024):
    n = similar.shape[0]
    in_feats = inputs.shape[1]
    out_feats = weight.shape[0]

    n_rows = _ceil_to(n, block_rows)
    n_cols = _ceil_to(n, 128)
    in_p = _ceil_to(in_feats, 128)
    out_p = _ceil_to(out_feats, 128)

    sim = similar
    if (n_rows, n_cols) != similar.shape:
        sim = jnp.pad(similar, ((0, n_rows - n), (0, n_cols - n)))
    x = inputs
    if (n_cols, in_p) != inputs.shape:
        x = jnp.pad(inputs, ((0, n_cols - n), (0, in_p - in_feats)))
    w_t = weight.T
    if (in_p, out_p) != w_t.shape:
        w_t = jnp.pad(w_t, ((0, in_p - in_feats), (0, out_p - out_feats)))
    b = bias
    if out_p != b.shape[0]:
        b = jnp.pad(b, (0, out_p - out_feats))

    b = b.reshape(1, out_p).astype(jnp.float32)

    grid = (n_rows // block_rows,)
    out = pl.pallas_call(
        _gcn_block_kernel,
        out_shape=jax.ShapeDtypeStruct((n_rows, out_p), jnp.float32),
        grid=grid,
        in_specs=[
            pl.BlockSpec((block_rows, n_cols), lambda i: (i, 0)),
            pl.BlockSpec((n_cols, in_p), lambda i: (0, 0)),
            pl.BlockSpec((in_p, out_p), lambda i: (0, 0)),
            pl.BlockSpec((1, out_p), lambda i: (0, 0)),
        ],
        out_specs=pl.BlockSpec((block_rows, out_p), lambda i: (i, 0)),
        compiler_params=pltpu.CompilerParams(
            dimension_semantics=("parallel",),
            vmem_limit_bytes=60 * 1024 * 1024,
        ),
    )(sim, x, w_t, b)

    if (n_rows, out_p) != (n, out_feats):
        out = out[:n, :out_feats]
    return out


def kernel(similar, inputs, weight, bias):
    return _gcn_layer(similar, inputs, weight, bias)


# speedup vs baseline: 1.0444x; 1.0444x over previous
import functools

import jax
import jax.numpy as jnp
from jax.experimental import pallas as pl
from jax.experimental.pallas import tpu as pltpu


def _ceil_to(x, m):
    return ((x + m - 1) // m) * m


def _gcn_block_kernel(sim_ref, x_ref, w_ref, b_ref, o_ref):
    h = jnp.dot(sim_ref[...], x_ref[...], preferred_element_type=jnp.float32)
    y = jnp.dot(h, w_ref[...], preferred_element_type=jnp.float32)
    o_ref[...] = jnp.maximum(y + b_ref[...], 0.0)


@functools.partial(jax.jit, static_argnames=("block_rows",))
def _gcn_layer(similar, inputs, weight, bias, *, block_rows=512):
    n = similar.shape[0]
    in_feats = inputs.shape[1]
    out_feats = weight.shape[0]

    n_rows = _ceil_to(n, block_rows)
    n_cols = _ceil_to(n, 128)
    in_p = _ceil_to(in_feats, 128)
    out_p = _ceil_to(out_feats, 128)

    sim = similar
    if (n_rows, n_cols) != similar.shape:
        sim = jnp.pad(similar, ((0, n_rows - n), (0, n_cols - n)))
    x = inputs
    if (n_cols, in_p) != inputs.shape:
        x = jnp.pad(inputs, ((0, n_cols - n), (0, in_p - in_feats)))
    w_t = weight.T
    if (in_p, out_p) != w_t.shape:
        w_t = jnp.pad(w_t, ((0, in_p - in_feats), (0, out_p - out_feats)))
    b = bias
    if out_p != b.shape[0]:
        b = jnp.pad(b, (0, out_p - out_feats))

    b = b.reshape(1, out_p).astype(jnp.float32)

    grid = (n_rows // block_rows,)
    out = pl.pallas_call(
        _gcn_block_kernel,
        out_shape=jax.ShapeDtypeStruct((n_rows, out_p), jnp.float32),
        grid=grid,
        in_specs=[
            pl.BlockSpec((block_rows, n_cols), lambda i: (i, 0)),
            pl.BlockSpec((n_cols, in_p), lambda i: (0, 0)),
            pl.BlockSpec((in_p, out_p), lambda i: (0, 0)),
            pl.BlockSpec((1, out_p), lambda i: (0, 0)),
        ],
        out_specs=pl.BlockSpec((block_rows, out_p), lambda i: (i, 0)),
        compiler_params=pltpu.CompilerParams(
            dimension_semantics=("arbitrary",),
            vmem_limit_bytes=60 * 1024 * 1024,
        ),
    )(sim, x, w_t, b)

    if (n_rows, out_p) != (n, out_feats):
        out = out[:n, :out_feats]
    return out


def kernel(similar, inputs, weight, bias):
    return _gcn_layer(similar, inputs, weight, bias)
